# baseline (speedup 1.0000x reference)
import jax
import jax.numpy as jnp
from jax.experimental import pallas as pl
from jax.experimental.pallas import tpu as pltpu


N = 10000
NHID = 128
NEMBED = 128
NCLASS = 40
BM = 200
NBUF = 5
NCHUNKS = N // BM
NROUNDS = NCHUNKS // NBUF


def _fused_body(adj_hbm, xf_ref, w1t_ref, w2t_ref, wmt_ref, b_ref, out_ref,
                bufs, sems):
    def copy_op(chunk, slot):
        return pltpu.make_async_copy(
            adj_hbm.at[pl.ds(chunk * BM, BM), :],
            bufs.at[slot],
            sems.at[slot],
        )

    for slot in range(NBUF):
        copy_op(slot, slot).start()

    xf = xf_ref[...]
    w1t = w1t_ref[...]
    w2t = w2t_ref[...]
    wmt = wmt_ref[...]
    b = b_ref[...]

    def round_step(r, carry):
        for slot in range(NBUF):
            i = r * NBUF + slot
            copy_op(i, slot).wait()
            adj_blk = bufs[slot]
            deg = jnp.sum(adj_blk, axis=1, keepdims=True)
            neigh = jnp.dot(adj_blk, xf,
                            preferred_element_type=jnp.float32)

            @pl.when(i + NBUF < NCHUNKS)
            def _():
                copy_op(i + NBUF, slot).start()

            neigh = neigh / (deg + 1.0)
            xi = xf_ref[pl.ds(i * BM, BM), :]
            h = (jnp.dot(xi, w1t, preferred_element_type=jnp.float32) +
                 jnp.dot(neigh, w2t, preferred_element_type=jnp.float32))
            h = jnp.maximum(h, 0.0)
            out_ref[pl.ds(i * BM, BM), :] = (
                jnp.dot(h, wmt, preferred_element_type=jnp.float32) + b)
        return carry

    jax.lax.fori_loop(0, NROUNDS, round_step, 0)


@jax.jit
def kernel(x, adj, W_proj, W_mlp, b_mlp):
    w1t = W_proj[:, :NHID].T
    w2t = W_proj[:, NHID:].T
    wmt = W_mlp.T
    b2 = b_mlp.reshape(1, NCLASS)

    out = pl.pallas_call(
        _fused_body,
        in_specs=[
            pl.BlockSpec(memory_space=pltpu.MemorySpace.HBM),
            pl.BlockSpec(memory_space=pltpu.MemorySpace.VMEM),
            pl.BlockSpec(memory_space=pltpu.MemorySpace.VMEM),
            pl.BlockSpec(memory_space=pltpu.MemorySpace.VMEM),
            pl.BlockSpec(memory_space=pltpu.MemorySpace.VMEM),
            pl.BlockSpec(memory_space=pltpu.MemorySpace.VMEM),
        ],
        out_specs=pl.BlockSpec(memory_space=pltpu.MemorySpace.VMEM),
        out_shape=jax.ShapeDtypeStruct((N, NCLASS), jnp.float32),
        scratch_shapes=[
            pltpu.VMEM((NBUF, BM, N), jnp.float32),
            pltpu.SemaphoreType.DMA((NBUF,)),
        ],
    )(adj, x, w1t, w2t, wmt, b2)
    return out


# speedup vs baseline: 1.0296x; 1.0296x over previous
import jax
import jax.numpy as jnp
from jax.experimental import pallas as pl
from jax.experimental.pallas import tpu as pltpu


N = 10000
NHID = 128
NEMBED = 128
NCLASS = 40
BM = 400
NBUF = 3
NCHUNKS = N // BM


def _fused_body(adj_hbm, xf_ref, w1t_ref, w2t_ref, wmt_ref, b_ref, out_ref,
                bufs, sems):
    def copy_op(chunk, slot):
        return pltpu.make_async_copy(
            adj_hbm.at[pl.ds(chunk * BM, BM), :],
            bufs.at[slot],
            sems.at[slot],
        )

    for slot in range(NBUF):
        copy_op(slot, slot).start()

    xf = xf_ref[...]
    w1t = w1t_ref[...]
    w2t = w2t_ref[...]
    wmt = wmt_ref[...]
    b = b_ref[...]

    def step(i, carry):
        slot = jax.lax.rem(i, NBUF)
        copy_op(i, slot).wait()
        adj_blk = bufs[slot]
        deg = jnp.sum(adj_blk, axis=1, keepdims=True)
        neigh = jnp.dot(adj_blk, xf,
                        preferred_element_type=jnp.float32)

        @pl.when(i + NBUF < NCHUNKS)
        def _():
            copy_op(i + NBUF, slot).start()

        neigh = neigh / (deg + 1.0)
        xi = xf_ref[pl.ds(i * BM, BM), :]
        h = (jnp.dot(xi, w1t, preferred_element_type=jnp.float32) +
             jnp.dot(neigh, w2t, preferred_element_type=jnp.float32))
        h = jnp.maximum(h, 0.0)
        out_ref[pl.ds(i * BM, BM), :] = (
            jnp.dot(h, wmt, preferred_element_type=jnp.float32) + b)
        return carry

    jax.lax.fori_loop(0, NCHUNKS, step, 0)


@jax.jit
def kernel(x, adj, W_proj, W_mlp, b_mlp):
    w1t = W_proj[:, :NHID].T
    w2t = W_proj[:, NHID:].T
    wmt = W_mlp.T
    b2 = b_mlp.reshape(1, NCLASS)

    out = pl.pallas_call(
        _fused_body,
        in_specs=[
            pl.BlockSpec(memory_space=pltpu.MemorySpace.HBM),
            pl.BlockSpec(memory_space=pltpu.MemorySpace.VMEM),
            pl.BlockSpec(memory_space=pltpu.MemorySpace.VMEM),
            pl.BlockSpec(memory_space=pltpu.MemorySpace.VMEM),
            pl.BlockSpec(memory_space=pltpu.MemorySpace.VMEM),
            pl.BlockSpec(memory_space=pltpu.MemorySpace.VMEM),
        ],
        out_specs=pl.BlockSpec(memory_space=pltpu.MemorySpace.VMEM),
        out_shape=jax.ShapeDtypeStruct((N, NCLASS), jnp.float32),
        scratch_shapes=[
            pltpu.VMEM((NBUF, BM, N), jnp.float32),
            pltpu.SemaphoreType.DMA((NBUF,)),
        ],
        compiler_params=pltpu.CompilerParams(
            vmem_limit_bytes=64 * 1024 * 1024),
    )(adj, x, w1t, w2t, wmt, b2)
    return out


# speedup vs baseline: 1.0657x; 1.0350x over previous
import jax
import jax.numpy as jnp
from jax.experimental import pallas as pl


N = 10000
NHID = 128
NEMBED = 128
NCLASS = 40
BM = 400


def _fused_body(adj_ref, xf_ref, w1t_ref, w2t_ref, wmt_ref, b_ref, out_ref):
    i = pl.program_id(0)
    adj_blk = adj_ref[...]
    deg = jnp.sum(adj_blk, axis=1, keepdims=True)
    neigh = jnp.dot(adj_blk, xf_ref[...],
                    preferred_element_type=jnp.float32)
    neigh = neigh / (deg + 1.0)
    xi = xf_ref[pl.ds(i * BM, BM), :]
    h = (jnp.dot(xi, w1t_ref[...], preferred_element_type=jnp.float32) +
         jnp.dot(neigh, w2t_ref[...], preferred_element_type=jnp.float32))
    h = jnp.maximum(h, 0.0)
    out_ref[...] = (jnp.dot(h, wmt_ref[...],
                            preferred_element_type=jnp.float32) +
                    b_ref[...])


@jax.jit
def kernel(x, adj, W_proj, W_mlp, b_mlp):
    w1t = W_proj[:, :NHID].T
    w2t = W_proj[:, NHID:].T
    wmt = W_mlp.T
    b2 = b_mlp.reshape(1, NCLASS)

    grid = (N // BM,)
    out = pl.pallas_call(
        _fused_body,
        grid=grid,
        in_specs=[
            pl.BlockSpec((BM, N), lambda i: (i, 0)),
            pl.BlockSpec((N, NHID), lambda i: (0, 0)),
            pl.BlockSpec((NHID, NEMBED), lambda i: (0, 0)),
            pl.BlockSpec((NHID, NEMBED), lambda i: (0, 0)),
            pl.BlockSpec((NEMBED, NCLASS), lambda i: (0, 0)),
            pl.BlockSpec((1, NCLASS), lambda i: (0, 0)),
        ],
        out_specs=pl.BlockSpec((BM, NCLASS), lambda i: (i, 0)),
        out_shape=jax.ShapeDtypeStruct((N, NCLASS), jnp.float32),
    )(adj, x, w1t, w2t, wmt, b2)
    return out
